# baseline (speedup 1.0000x reference)
import functools

import jax
import jax.numpy as jnp
from jax import lax
from jax.experimental import pallas as pl
from jax.experimental.pallas import tpu as pltpu
from jax.experimental.pallas import tpu_sc as plsc

B, S, V, H = 4, 2048, 128100, 1024
NT = B * S
LN_EPS = 1e-07

_info = plsc.get_sparse_core_info()
NC, NS = _info.num_cores, _info.num_subcores
NW = NC * NS
NSLICE = 4
QS = S // NSLICE
TS = B * QS
WPB = NW // B
T_PER_W = TS // NW
CHUNK = 32
N_CHUNKS = T_PER_W // CHUNK


def _sc_gather_slice(idx_grouped, table):
    mesh = plsc.VectorSubcoreMesh(core_axis_name="c", subcore_axis_name="s")

    @functools.partial(
        pl.kernel,
        mesh=mesh,
        out_type=jax.ShapeDtypeStruct((TS, H), jnp.bfloat16),
        scratch_types=[
            pltpu.VMEM((N_CHUNKS, CHUNK), jnp.int32),
            pltpu.VMEM((N_CHUNKS, CHUNK, H), jnp.float32),
            pltpu.VMEM((N_CHUNKS, CHUNK, H), jnp.bfloat16),
            pltpu.SemaphoreType.DMA,
            pltpu.SemaphoreType.DMA,
            pltpu.SemaphoreType.DMA,
            pltpu.SemaphoreType.DMA,
        ],
        compiler_params=pltpu.CompilerParams(
            use_tc_tiling_on_sc=False, needs_layout_passes=False),
    )
    def k(idx_hbm, table_hbm, out_hbm, idx_v, rows_v, bf_v, g0, g1, w0, w1):
        wid = lax.axis_index("s") * NC + lax.axis_index("c")
        base = wid * T_PER_W
        gsem = (g0, g1)
        wsem = (w0, w1)
        ii2 = lax.iota(jnp.int32, 16) * 2
        pltpu.sync_copy(idx_hbm.at[wid], idx_v)
        gh = [pltpu.async_copy(table_hbm.at[idx_v.at[c]], rows_v.at[c], gsem[c])
              for c in range(N_CHUNKS)]
        wh = []
        for c in range(N_CHUNKS):
            gh[c].wait()
            src = rows_v.at[c]
            dst = bf_v.at[c]

            def row_body(r, _, src=src, dst=dst):
                rr = jnp.full((16,), r, jnp.int32)

                def grp_body(g, _):
                    ev = plsc.load_gather(src, [rr, g * 32 + ii2])
                    od = plsc.load_gather(src, [rr, g * 32 + ii2 + 1])
                    dst[r, pl.ds(g * 32, 32)] = plsc.pack(
                        ev, od, format=plsc.PackFormat.INTERLEAVED)
                    return 0
                lax.fori_loop(0, H // 32, grp_body, 0, unroll=4)
                return 0
            lax.fori_loop(0, CHUNK, row_body, 0)
            wh.append(pltpu.async_copy(
                dst,
                out_hbm.at[pl.ds(base + c * CHUNK, CHUNK)],
                wsem[c]))
        for h in wh:
            h.wait()

    return k(idx_grouped, table)


ROWS_BLK = 256
PB = QS // ROWS_BLK


def _ln_body(g_ref, p_ref, s_ref, b_ref, *rest):
    o_ref = rest[-1]
    x = g_ref[...].astype(jnp.float32) + p_ref[...]
    mean = jnp.mean(x, axis=-1, keepdims=True)
    var = jnp.mean(jnp.square(x - mean), axis=-1, keepdims=True)
    normed = (x - mean) * lax.rsqrt(var + LN_EPS)
    o_ref[...] = normed * s_ref[...] + b_ref[...]


def _tc_add_ln_slice(gathered, pos, scale, bias, buf, s):
    operands = [gathered, pos, scale, bias]
    in_specs = [
        pl.BlockSpec((ROWS_BLK, H), lambda i, j: (j * PB + i, 0)),
        pl.BlockSpec((ROWS_BLK, H), lambda i, j, s=s: (s * PB + i, 0)),
        pl.BlockSpec((1, H), lambda i, j: (0, 0)),
        pl.BlockSpec((1, H), lambda i, j: (0, 0)),
    ]
    aliases = {}
    if buf is not None:
        operands.append(buf)
        in_specs.append(pl.BlockSpec(memory_space=pl.ANY))
        aliases = {4: 0}
    return pl.pallas_call(
        _ln_body,
        grid=(PB, B),
        in_specs=in_specs,
        out_specs=pl.BlockSpec(
            (ROWS_BLK, H),
            lambda i, j, s=s: (j * (S // ROWS_BLK) + s * PB + i, 0)),
        out_shape=jax.ShapeDtypeStruct((NT, H), jnp.float32),
        input_output_aliases=aliases,
    )(*operands)


def kernel(input_ids, word_embeddings, position_embeddings, ln_scale, ln_bias):
    ids5 = input_ids.astype(jnp.int32).reshape(B, NSLICE, WPB, N_CHUNKS, CHUNK)
    scale2 = ln_scale.reshape(1, H)
    bias2 = ln_bias.reshape(1, H)
    gathered = [
        _sc_gather_slice(ids5[:, s].reshape(NW, N_CHUNKS, CHUNK),
                         word_embeddings)
        for s in range(NSLICE)
    ]
    buf = None
    for s in range(NSLICE):
        buf = _tc_add_ln_slice(gathered[s], position_embeddings,
                               scale2, bias2, buf, s)
    return buf.reshape(B, S, H)


# speedup vs baseline: 7.3016x; 7.3016x over previous
import functools

import jax
import jax.numpy as jnp
from jax import lax
from jax.experimental import pallas as pl
from jax.experimental.pallas import tpu as pltpu
from jax.experimental.pallas import tpu_sc as plsc

B, S, V, H = 4, 2048, 128100, 1024
NT = B * S
LN_EPS = 1e-07

_info = plsc.get_sparse_core_info()
NC, NS = _info.num_cores, _info.num_subcores
NW = NC * NS
NSLICE = 8
QS = S // NSLICE
TS = B * QS
WPB = NW // B
T_PER_W = TS // NW
CHUNK = T_PER_W


def _sc_gather_slice(idx_flat, table, s):
    mesh = plsc.VectorSubcoreMesh(core_axis_name="c", subcore_axis_name="s")

    @functools.partial(
        pl.kernel,
        mesh=mesh,
        out_type=jax.ShapeDtypeStruct((TS, H), jnp.float32),
        scratch_types=[
            pltpu.VMEM((CHUNK,), jnp.int32),
            pltpu.VMEM((CHUNK, H), jnp.float32),
            pltpu.SemaphoreType.DMA,
            pltpu.SemaphoreType.DMA,
        ],
    )
    def k(idx_hbm, table_hbm, out_hbm, idx_v, rows_v, gsem, wsem):
        wid = lax.axis_index("s") * NC + lax.axis_index("c")
        b = wid // WPB
        w8 = wid % WPB
        pltpu.sync_copy(
            idx_hbm.at[pl.ds(b * S + s * QS + w8 * CHUNK, CHUNK)], idx_v)
        pltpu.async_copy(table_hbm.at[idx_v], rows_v, gsem).wait()
        pltpu.async_copy(
            rows_v, out_hbm.at[pl.ds(wid * CHUNK, CHUNK)], wsem).wait()

    return k(idx_flat, table)


ROWS_BLK = QS


def _ln_body(g_ref, p_ref, s_ref, b_ref, *rest):
    o_ref = rest[-1]
    x = g_ref[...] + p_ref[...]
    mean = jnp.mean(x, axis=-1, keepdims=True)
    var = jnp.mean(jnp.square(x - mean), axis=-1, keepdims=True)
    normed = (x - mean) * lax.rsqrt(var + LN_EPS)
    o_ref[...] = normed * s_ref[...] + b_ref[...]


def _tc_add_ln_slice(gathered, pos, scale, bias, buf, s):
    operands = [gathered, pos, scale, bias]
    in_specs = [
        pl.BlockSpec((ROWS_BLK, H), lambda j: (j, 0)),
        pl.BlockSpec((ROWS_BLK, H), lambda j, s=s: (s, 0)),
        pl.BlockSpec((1, H), lambda j: (0, 0)),
        pl.BlockSpec((1, H), lambda j: (0, 0)),
    ]
    aliases = {}
    if buf is not None:
        operands.append(buf)
        in_specs.append(pl.BlockSpec(memory_space=pl.ANY))
        aliases = {4: 0}
    return pl.pallas_call(
        _ln_body,
        grid=(B,),
        in_specs=in_specs,
        out_specs=pl.BlockSpec(
            (ROWS_BLK, H), lambda j, s=s: (j * NSLICE + s, 0)),
        out_shape=jax.ShapeDtypeStruct((NT, H), jnp.float32),
        input_output_aliases=aliases,
    )(*operands)


def kernel(input_ids, word_embeddings, position_embeddings, ln_scale, ln_bias):
    idx_flat = input_ids.astype(jnp.int32).reshape(NT)
    scale2 = ln_scale.reshape(1, H)
    bias2 = ln_bias.reshape(1, H)
    gathered = [_sc_gather_slice(idx_flat, word_embeddings, s)
                for s in range(NSLICE)]
    buf = None
    for s in range(NSLICE):
        buf = _tc_add_ln_slice(gathered[s], position_embeddings,
                               scale2, bias2, buf, s)
    return buf.reshape(B, S, H)


# speedup vs baseline: 8.5663x; 1.1732x over previous
import functools

import jax
import jax.numpy as jnp
from jax import lax
from jax.experimental import pallas as pl
from jax.experimental.pallas import tpu as pltpu
from jax.experimental.pallas import tpu_sc as plsc

B, S, V, H = 4, 2048, 128100, 1024
NT = B * S
LN_EPS = 1e-07

_info = plsc.get_sparse_core_info()
NC, NS = _info.num_cores, _info.num_subcores
NW = NC * NS
NSLICE = 4
QS = S // NSLICE
TS = B * QS
WPB = NW // B
T_PER_W = TS // NW
CHUNK = 32
N_CHUNKS = T_PER_W // CHUNK


def _sc_gather_slice(idx_flat, table, s):
    mesh = plsc.VectorSubcoreMesh(core_axis_name="c", subcore_axis_name="s")

    @functools.partial(
        pl.kernel,
        mesh=mesh,
        out_type=jax.ShapeDtypeStruct((TS, H), jnp.float32),
        scratch_types=[
            pltpu.VMEM((T_PER_W,), jnp.int32),
            pltpu.VMEM((N_CHUNKS, CHUNK, H), jnp.float32),
            pltpu.SemaphoreType.DMA,
            pltpu.SemaphoreType.DMA,
            pltpu.SemaphoreType.DMA,
            pltpu.SemaphoreType.DMA,
        ],
    )
    def k(idx_hbm, table_hbm, out_hbm, idx_v, rows_v, g0, g1, w0, w1):
        wid = lax.axis_index("s") * NC + lax.axis_index("c")
        b = wid // WPB
        w8 = wid % WPB
        base = b * S + s * QS + w8 * T_PER_W
        gsem = (g0, g1)
        wsem = (w0, w1)
        pltpu.sync_copy(idx_hbm.at[pl.ds(base, T_PER_W)], idx_v)
        gh = [pltpu.async_copy(
                  table_hbm.at[idx_v.at[pl.ds(c * CHUNK, CHUNK)]],
                  rows_v.at[c], gsem[c])
              for c in range(N_CHUNKS)]
        wh = []
        for c in range(N_CHUNKS):
            gh[c].wait()
            wh.append(pltpu.async_copy(
                rows_v.at[c],
                out_hbm.at[pl.ds(wid * T_PER_W + c * CHUNK, CHUNK)],
                wsem[c]))
        for h in wh:
            h.wait()

    return k(idx_flat, table)


ROWS_BLK = 256
PB = QS // ROWS_BLK


def _ln_body(g_ref, p_ref, s_ref, b_ref, *rest):
    o_ref = rest[-1]
    x = g_ref[...] + p_ref[...]
    mean = jnp.mean(x, axis=-1, keepdims=True)
    var = jnp.mean(jnp.square(x - mean), axis=-1, keepdims=True)
    normed = (x - mean) * lax.rsqrt(var + LN_EPS)
    o_ref[...] = normed * s_ref[...] + b_ref[...]


def _tc_add_ln_slice(gathered, pos, scale, bias, buf, s):
    operands = [gathered, pos, scale, bias]
    in_specs = [
        pl.BlockSpec((ROWS_BLK, H), lambda i, j: (j * PB + i, 0)),
        pl.BlockSpec((ROWS_BLK, H), lambda i, j, s=s: (s * PB + i, 0)),
        pl.BlockSpec((1, H), lambda i, j: (0, 0)),
        pl.BlockSpec((1, H), lambda i, j: (0, 0)),
    ]
    aliases = {}
    if buf is not None:
        operands.append(buf)
        in_specs.append(pl.BlockSpec(memory_space=pl.ANY))
        aliases = {4: 0}
    return pl.pallas_call(
        _ln_body,
        grid=(PB, B),
        in_specs=in_specs,
        out_specs=pl.BlockSpec(
            (ROWS_BLK, H),
            lambda i, j, s=s: (j * (S // ROWS_BLK) + s * PB + i, 0)),
        out_shape=jax.ShapeDtypeStruct((NT, H), jnp.float32),
        input_output_aliases=aliases,
    )(*operands)


def kernel(input_ids, word_embeddings, position_embeddings, ln_scale, ln_bias):
    idx_flat = input_ids.astype(jnp.int32).reshape(NT)
    scale2 = ln_scale.reshape(1, H)
    bias2 = ln_bias.reshape(1, H)
    gathered = [_sc_gather_slice(idx_flat, word_embeddings, s)
                for s in range(NSLICE)]
    buf = None
    for s in range(NSLICE):
        buf = _tc_add_ln_slice(gathered[s], position_embeddings,
                               scale2, bias2, buf, s)
    return buf.reshape(B, S, H)


# speedup vs baseline: 8.8703x; 1.0355x over previous
import functools

import jax
import jax.numpy as jnp
from jax import lax
from jax.experimental import pallas as pl
from jax.experimental.pallas import tpu as pltpu
from jax.experimental.pallas import tpu_sc as plsc

B, S, V, H = 4, 2048, 128100, 1024
NT = B * S
LN_EPS = 1e-07

_info = plsc.get_sparse_core_info()
NC, NS = _info.num_cores, _info.num_subcores
NW = NC * NS
NSLICE = 4
QS = S // NSLICE
TS = B * QS
WPB = NW // B
T_PER_W = TS // NW
CHUNK = 32
N_CHUNKS = T_PER_W // CHUNK


def _sc_gather_slice(idx_flat, table, s):
    mesh = plsc.VectorSubcoreMesh(core_axis_name="c", subcore_axis_name="s")

    @functools.partial(
        pl.kernel,
        mesh=mesh,
        out_type=jax.ShapeDtypeStruct((TS, H), jnp.float32),
        scratch_types=[
            pltpu.VMEM((T_PER_W,), jnp.int32),
            pltpu.VMEM((N_CHUNKS, CHUNK, H), jnp.float32),
            pltpu.SemaphoreType.DMA,
            pltpu.SemaphoreType.DMA,
            pltpu.SemaphoreType.DMA,
            pltpu.SemaphoreType.DMA,
        ],
    )
    def k(idx_hbm, table_hbm, out_hbm, idx_v, rows_v, g0, g1, w0, w1):
        wid = lax.axis_index("s") * NC + lax.axis_index("c")
        b = wid // WPB
        w8 = wid % WPB
        base = b * S + s * QS + w8 * T_PER_W
        gsem = (g0, g1)
        wsem = (w0, w1)
        pltpu.sync_copy(idx_hbm.at[pl.ds(base, T_PER_W)], idx_v)
        gh = [pltpu.async_copy(
                  table_hbm.at[idx_v.at[pl.ds(c * CHUNK, CHUNK)]],
                  rows_v.at[c], gsem[c])
              for c in range(N_CHUNKS)]
        wh = []
        for c in range(N_CHUNKS):
            gh[c].wait()
            wh.append(pltpu.async_copy(
                rows_v.at[c],
                out_hbm.at[pl.ds(wid * T_PER_W + c * CHUNK, CHUNK)],
                wsem[c]))
        for h in wh:
            h.wait()

    return k(idx_flat, table)


ROWS_BLK = 512
PB = QS // ROWS_BLK


def _ln_body(g_ref, p_ref, s_ref, b_ref, *rest):
    o_ref = rest[-1]
    x = g_ref[...] + p_ref[...]
    mean = jnp.mean(x, axis=-1, keepdims=True)
    var = jnp.mean(jnp.square(x - mean), axis=-1, keepdims=True)
    normed = (x - mean) * lax.rsqrt(var + LN_EPS)
    o_ref[...] = normed * s_ref[...] + b_ref[...]


def _tc_add_ln_slice(gathered, pos, scale, bias, buf, s):
    operands = [gathered, pos, scale, bias]
    in_specs = [
        pl.BlockSpec((ROWS_BLK, H), lambda i, j: (j * PB + i, 0)),
        pl.BlockSpec((ROWS_BLK, H), lambda i, j, s=s: (s * PB + i, 0)),
        pl.BlockSpec((1, H), lambda i, j: (0, 0)),
        pl.BlockSpec((1, H), lambda i, j: (0, 0)),
    ]
    aliases = {}
    if buf is not None:
        operands.append(buf)
        in_specs.append(pl.BlockSpec(memory_space=pl.ANY))
        aliases = {4: 0}
    return pl.pallas_call(
        _ln_body,
        grid=(PB, B),
        in_specs=in_specs,
        out_specs=pl.BlockSpec(
            (ROWS_BLK, H),
            lambda i, j, s=s: (j * (S // ROWS_BLK) + s * PB + i, 0)),
        out_shape=jax.ShapeDtypeStruct((NT, H), jnp.float32),
        input_output_aliases=aliases,
    )(*operands)


def kernel(input_ids, word_embeddings, position_embeddings, ln_scale, ln_bias):
    idx_flat = input_ids.astype(jnp.int32).reshape(NT)
    scale2 = ln_scale.reshape(1, H)
    bias2 = ln_bias.reshape(1, H)
    gathered = [_sc_gather_slice(idx_flat, word_embeddings, s)
                for s in range(NSLICE)]
    buf = None
    for s in range(NSLICE):
        buf = _tc_add_ln_slice(gathered[s], position_embeddings,
                               scale2, bias2, buf, s)
    return buf.reshape(B, S, H)
